# Initial kernel scaffold; baseline (speedup 1.0000x reference)
import jax
import jax.numpy as jnp
from jax.experimental import pallas as pl


def kernel(x, edge_index, edge_attr, edge_gate_type, batch, global_features, params):
    raise NotImplementedError("write your pallas kernel here")


# speedup vs baseline: 1.2854x; 1.2854x over previous
import functools

import jax
import jax.numpy as jnp
from jax import lax
from jax.experimental import pallas as pl
from jax.experimental.pallas import tpu as pltpu

N = 10000
E = 320000
H = 256
NG = 64
NC = 9
NGT = 20

_BM = 2000


def _ln(x, g, b):
    m = jnp.mean(x, axis=-1, keepdims=True)
    v = jnp.mean((x - m) ** 2, axis=-1, keepdims=True)
    return g * (x - m) * lax.rsqrt(v + 1e-5) + b


def _dot(a, b):
    return jnp.dot(a, b, preferred_element_type=jnp.float32)


def _embed_body(x_ref, w_ref, b_ref, g_ref, bb_ref, o_ref):
    h = jnp.maximum(_dot(x_ref[...], w_ref[...]) + b_ref[...], 0.0)
    o_ref[...] = _ln(h, g_ref[...], bb_ref[...])


def _embed(x, w, b, g, bb):
    nf = x.shape[1]
    return pl.pallas_call(
        _embed_body,
        grid=(N // _BM,),
        in_specs=[
            pl.BlockSpec((_BM, nf), lambda i: (i, 0)),
            pl.BlockSpec((nf, H), lambda i: (0, 0)),
            pl.BlockSpec((1, H), lambda i: (0, 0)),
            pl.BlockSpec((1, H), lambda i: (0, 0)),
            pl.BlockSpec((1, H), lambda i: (0, 0)),
        ],
        out_specs=pl.BlockSpec((_BM, H), lambda i: (i, 0)),
        out_shape=jax.ShapeDtypeStruct((N, H), jnp.float32),
    )(x, w, b.reshape(1, H), g.reshape(1, H), bb.reshape(1, H))


def _z_body(ea_ref, gt_ref, wea_ref, wg_ref, b_ref, o_ref):
    gt = gt_ref[0, 0, :]
    oh = (gt[:, None] == lax.broadcasted_iota(jnp.int32, (_BM, 32), 1)
          ).astype(jnp.float32)
    z = _dot(ea_ref[...], wea_ref[...]) + _dot(oh, wg_ref[...])
    o_ref[...] = z + b_ref[...]


def _z(ea, gt3, wea, wg, b):
    ef = ea.shape[1]
    return pl.pallas_call(
        _z_body,
        grid=(E // _BM,),
        in_specs=[
            pl.BlockSpec((_BM, ef), lambda i: (i, 0)),
            pl.BlockSpec((1, 1, _BM), lambda i: (i, 0, 0)),
            pl.BlockSpec((ef, H), lambda i: (0, 0)),
            pl.BlockSpec((32, H), lambda i: (0, 0)),
            pl.BlockSpec((1, H), lambda i: (0, 0)),
        ],
        out_specs=pl.BlockSpec((_BM, H), lambda i: (i, 0)),
        out_shape=jax.ShapeDtypeStruct((E, H), jnp.float32),
    )(ea, gt3, wea, wg, b.reshape(1, H))


def _lin_body(x_ref, w_ref, o_ref):
    o_ref[...] = _dot(x_ref[...], w_ref[...])


def _lin(x, w):
    return pl.pallas_call(
        _lin_body,
        grid=(N // _BM,),
        in_specs=[
            pl.BlockSpec((_BM, H), lambda i: (i, 0)),
            pl.BlockSpec((H, H), lambda i: (0, 0)),
        ],
        out_specs=pl.BlockSpec((_BM, H), lambda i: (i, 0)),
        out_shape=jax.ShapeDtypeStruct((N, H), jnp.float32),
    )(x, w)


def _post_body(h_ref, a1_ref, deg_ref, mw2_ref, mb2_ref, w1h_ref, w1a_ref,
               ub1_ref, w2_ref, ub2_ref, ng_ref, nb_ref, o_ref):
    agg = _dot(a1_ref[...], mw2_ref[...]) + deg_ref[0, 0, :][:, None] * mb2_ref[...]
    t = jnp.maximum(_dot(h_ref[...], w1h_ref[...]) + _dot(agg, w1a_ref[...])
                    + ub1_ref[...], 0.0)
    u = _dot(t, w2_ref[...]) + ub2_ref[...]
    u = _ln(u, ng_ref[...], nb_ref[...])
    o_ref[...] = h_ref[...] + u


def _post(h, agg1, deg3, L):
    v = lambda a: a.reshape(1, H)
    return pl.pallas_call(
        _post_body,
        grid=(N // _BM,),
        in_specs=[
            pl.BlockSpec((_BM, H), lambda i: (i, 0)),
            pl.BlockSpec((_BM, H), lambda i: (i, 0)),
            pl.BlockSpec((1, 1, _BM), lambda i: (i, 0, 0)),
            pl.BlockSpec((H, H), lambda i: (0, 0)),
            pl.BlockSpec((1, H), lambda i: (0, 0)),
            pl.BlockSpec((H, H), lambda i: (0, 0)),
            pl.BlockSpec((H, H), lambda i: (0, 0)),
            pl.BlockSpec((1, H), lambda i: (0, 0)),
            pl.BlockSpec((H, H), lambda i: (0, 0)),
            pl.BlockSpec((1, H), lambda i: (0, 0)),
            pl.BlockSpec((1, H), lambda i: (0, 0)),
            pl.BlockSpec((1, H), lambda i: (0, 0)),
        ],
        out_specs=pl.BlockSpec((_BM, H), lambda i: (i, 0)),
        out_shape=jax.ShapeDtypeStruct((N, H), jnp.float32),
    )(h, agg1, deg3, L['mW2'], v(L['mb2']), L['uW1'][:H], L['uW1'][H:],
      v(L['ub1']), L['uW2'], v(L['ub2']), v(L['ng']), v(L['nb']))


def _head_body(hm_ref, hx_ref, hs_ref, gf_ref, gw_ref, gb_ref, gg_ref, gbb_ref,
               w1m_ref, w1x_ref, w1s_ref, w1g_ref, cb1_ref, cg_ref, cbb_ref,
               w2_ref, cb2_ref, hw_ref, hb_ref, o_ref):
    g = jnp.maximum(_dot(gf_ref[...], gw_ref[...]) + gb_ref[...], 0.0)
    g = _ln(g, gg_ref[...], gbb_ref[...])
    c = jnp.maximum(
        _dot(hm_ref[...], w1m_ref[...]) + _dot(hx_ref[...], w1x_ref[...])
        + _dot(hs_ref[...], w1s_ref[...]) + _dot(g, w1g_ref[...]) + cb1_ref[...],
        0.0)
    c = _ln(c, cg_ref[...], cbb_ref[...])
    c = jnp.maximum(_dot(c, w2_ref[...]) + cb2_ref[...], 0.0)
    o_ref[...] = _dot(c, hw_ref[...]) + hb_ref[...]


def _head(hm, hx, hs, gf, p):
    gfd = gf.shape[1]
    h2 = 2 * H
    hwp = jnp.zeros((h2 // 2, 128), jnp.float32).at[:, :NC].set(p['hW'])
    hbp = jnp.zeros((1, 128), jnp.float32).at[0, :NC].set(p['hb'])
    w1 = p['cW1']
    full = lambda s: pl.BlockSpec(s, lambda: tuple(0 for _ in s))
    out = pl.pallas_call(
        _head_body,
        in_specs=[full((NG, H))] * 3 + [
            full((NG, gfd)), full((gfd, H)), full((1, H)), full((1, H)),
            full((1, H)), full((H, h2)), full((H, h2)), full((H, h2)),
            full((H, h2)), full((1, h2)), full((1, h2)), full((1, h2)),
            full((h2, H)), full((1, H)), full((H, 128)), full((1, 128)),
        ],
        out_specs=full((NG, 128)),
        out_shape=jax.ShapeDtypeStruct((NG, 128), jnp.float32),
    )(hm, hx, hs, gf, p['gW'], p['gb'].reshape(1, H), p['gg'].reshape(1, H),
      p['gbb'].reshape(1, H), w1[:H], w1[H:2 * H], w1[2 * H:3 * H],
      w1[3 * H:], p['cb1'].reshape(1, h2), p['cg'].reshape(1, h2),
      p['cbb'].reshape(1, h2), p['cW2'], p['cb2'].reshape(1, H), hwp, hbp)
    return out[:, :NC]


def kernel(x, edge_index, edge_attr, edge_gate_type, batch, global_features,
           params):
    p = params
    f32 = jnp.float32
    src = edge_index[0].astype(jnp.int32)
    dst = edge_index[1].astype(jnp.int32)
    gt3 = edge_gate_type.astype(jnp.int32).reshape(E // _BM, 1, _BM)

    h = _embed(x, p['node_W'], p['node_b'], p['node_g'], p['node_bb'])
    deg = jax.ops.segment_sum(jnp.ones((E,), f32), dst, num_segments=N)
    deg3 = deg.reshape(N // _BM, 1, _BM)

    for L in p['layers']:
        wg = jnp.zeros((32, H), f32).at[:NGT].set(L['ge'] @ L['mW1'][H:2 * H])
        z = _z(edge_attr, gt3, L['mW1'][2 * H:], wg, L['mb1'])
        a = _lin(h, L['mW1'][:H])
        m1 = jnp.maximum(a[src] + z, 0.0)
        agg1 = jax.ops.segment_sum(m1, dst, num_segments=N)
        h = _post(h, agg1, deg3, L)

    cnt = jax.ops.segment_sum(jnp.ones((N,), f32), batch, num_segments=NG)
    h_sum = jax.ops.segment_sum(h, batch, num_segments=NG)
    h_mean = h_sum / jnp.clip(cnt, 1.0)[:, None]
    h_max = jax.ops.segment_max(h, batch, num_segments=NG)
    return _head(h_mean, h_max, h_sum, global_features, p)
